# Initial kernel scaffold; baseline (speedup 1.0000x reference)
import jax
import jax.numpy as jnp
from jax.experimental import pallas as pl


def kernel(x, edge_index, edge_weight, Wz, bz, Lz_w, Lz_b, Wr, br, Lr_w, Lr_b, Wh, bh, Lh_w, Lh_b, att, lin_w, lin_b):
    raise NotImplementedError("write your pallas kernel here")


# speedup vs baseline: 15.7529x; 15.7529x over previous
import functools

import jax
import jax.numpy as jnp
from jax import lax
from jax.experimental import pallas as pl
from jax.experimental.pallas import tpu as pltpu
from jax.experimental.pallas import tpu_sc as plsc

N = 10000
E = 320000
F = 128
OC = 128
OS = 64

NC = 2
NS = 16
NW = NC * NS
NP = 10240
SL = NP // NS
EDEG = E // NS
EW_ = E // NW
C = 80
NCH = EW_ // C
L = 16

_MAGIC = 0x5F3759DF


def _rsqrt16(v):
    i = plsc.bitcast(v, jnp.int32)
    y = plsc.bitcast(jnp.int32(_MAGIC) - (i >> 1), jnp.float32)
    for _ in range(3):
        y = y * (1.5 - 0.5 * v * y * y)
    return y


def _sc_body(x_hbm, src_hbm, dst_hbm, ew_hbm,
             agg_hbm, dis_hbm,
             rows, disloc, srcc, dstc, ewc, normb, acc,
             degsh, dissh, aggsh, sem):
    cid = lax.axis_index("c")
    sid = lax.axis_index("s")
    wid = sid * NC + cid
    zero16 = jnp.zeros((L,), jnp.float32)

    def zrow(i, _):
        for k in range(F // L):
            rows[i, pl.ds(k * L, L)] = zero16
        return 0
    lax.fori_loop(0, C, zrow, 0)

    def zacc(i, _):
        acc[pl.ds(i * L, L)] = zero16
        return 0
    lax.fori_loop(0, SL // L, zacc, 0)

    pltpu.sync_copy(acc, degsh.at[pl.ds(sid * SL, SL)])
    for j in range(SL // C):
        pltpu.sync_copy(rows, aggsh.at[pl.ds(sid * SL + j * C, C), :])
    plsc.subcore_barrier()

    def degstep(i, _):
        base = sid * EDEG + i * C
        pltpu.sync_copy(dst_hbm.at[pl.ds(base, C)], dstc)
        pltpu.sync_copy(ew_hbm.at[pl.ds(base, C)], ewc)
        pltpu.sync_copy(ewc, degsh.at[dstc], add=True)
        return 0
    lax.fori_loop(0, EDEG // C, degstep, 0)
    plsc.subcore_barrier()

    pltpu.sync_copy(degsh.at[pl.ds(sid * SL, SL)], acc)

    def dstep(j, _):
        v = acc[pl.ds(j * L, L)] + 1.0
        acc[pl.ds(j * L, L)] = _rsqrt16(v)
        return 0
    lax.fori_loop(0, SL // L, dstep, 0)

    pltpu.sync_copy(acc, dissh.at[pl.ds(sid * SL, SL)])

    @pl.when(cid == 0)
    def _():
        pltpu.sync_copy(acc, dis_hbm.at[pl.ds(sid * SL, SL)])

    plsc.subcore_barrier()
    pltpu.sync_copy(dissh, disloc)

    def chunk(i, _):
        base = wid * EW_ + i * C
        pltpu.sync_copy(src_hbm.at[pl.ds(base, C)], srcc)
        pltpu.sync_copy(dst_hbm.at[pl.ds(base, C)], dstc)
        pltpu.sync_copy(ew_hbm.at[pl.ds(base, C)], ewc)
        pltpu.async_copy(x_hbm.at[srcc], rows, sem).wait()
        for g in range(C // L):
            s16 = srcc[pl.ds(g * L, L)]
            d16 = dstc[pl.ds(g * L, L)]
            w16 = ewc[pl.ds(g * L, L)]
            dis_s = plsc.load_gather(disloc, [s16])
            dis_d = plsc.load_gather(disloc, [d16])
            normb[pl.ds(g * L, L)] = dis_s * w16 * dis_d

        def scale(e, _):
            splat = plsc.load_gather(normb, [jnp.full((L,), e, jnp.int32)])
            for k in range(F // L):
                rows[e, pl.ds(k * L, L)] = rows[e, pl.ds(k * L, L)] * splat
            return 0
        lax.fori_loop(0, C, scale, 0)

        pltpu.sync_copy(rows, aggsh.at[dstc], add=True)
        return 0
    lax.fori_loop(0, NCH, chunk, 0)

    plsc.subcore_barrier()
    pltpu.sync_copy(aggsh.at[pl.ds(sid * SL, SL), :],
                    agg_hbm.at[cid, pl.ds(sid * SL, SL), :])


@jax.jit
def _sc_aggregate(x, src, dst, ew):
    return pl.kernel(
        _sc_body,
        out_type=[
            jax.ShapeDtypeStruct((NC, NP, F), jnp.float32),
            jax.ShapeDtypeStruct((NP,), jnp.float32),
        ],
        mesh=plsc.VectorSubcoreMesh(core_axis_name="c", subcore_axis_name="s"),
        compiler_params=pltpu.CompilerParams(needs_layout_passes=False),
        scratch_types=[
            pltpu.VMEM((C, F), jnp.float32),
            pltpu.VMEM((NP,), jnp.float32),
            pltpu.VMEM((C,), jnp.int32),
            pltpu.VMEM((C,), jnp.int32),
            pltpu.VMEM((C,), jnp.float32),
            pltpu.VMEM((C,), jnp.float32),
            pltpu.VMEM((SL,), jnp.float32),
            pltpu.VMEM_SHARED((NP,), jnp.float32),
            pltpu.VMEM_SHARED((NP,), jnp.float32),
            pltpu.VMEM_SHARED((NP, F), jnp.float32),
            pltpu.SemaphoreType.DMA,
        ],
    )(x, src, dst, ew)


BR = 256


def _tc_body(agg_ref, x_ref, dis_ref, wz_ref, bz_ref, lzt_ref, lzb_ref,
             wh_ref, bh_ref, lht_ref, lhb_ref, lint_ref, linb_ref, out_ref):
    dis = dis_ref[:]
    d2 = (dis * dis).reshape(BR, 1)
    a = agg_ref[0] + agg_ref[1] + x_ref[:] * d2
    tz = jnp.dot(a, wz_ref[:], preferred_element_type=jnp.float32) + bz_ref[:]
    z = jax.nn.sigmoid(
        jnp.dot(tz, lzt_ref[:], preferred_element_type=jnp.float32) + lzb_ref[:])
    th = jnp.dot(a, wh_ref[:], preferred_element_type=jnp.float32) + bh_ref[:]
    ht = jnp.tanh(
        jnp.dot(th, lht_ref[:], preferred_element_type=jnp.float32) + lhb_ref[:])
    h = jax.nn.relu((1.0 - z) * ht)
    out_ref[:, :] = (
        jnp.dot(h, lint_ref[:], preferred_element_type=jnp.float32) + linb_ref[:])


@jax.jit
def _tc_dense(agg, xp, dis, Wz, bz, LzT, Lz_b, Wh, bh, LhT, Lh_b, linT, lin_b):
    full = lambda shp: pl.BlockSpec(shp, lambda i: tuple(0 for _ in shp))
    return pl.pallas_call(
        _tc_body,
        grid=(NP // BR,),
        in_specs=[
            pl.BlockSpec((NC, BR, F), lambda i: (0, i, 0)),
            pl.BlockSpec((BR, F), lambda i: (i, 0)),
            pl.BlockSpec((BR,), lambda i: (i,)),
            full((F, OC)), full((OC,)), full((OC, OC)), full((OC,)),
            full((F, OC)), full((OC,)), full((OC, OC)), full((OC,)),
            full((OC, OS)), full((OS,)),
        ],
        out_specs=pl.BlockSpec((BR, OS), lambda i: (i, 0)),
        out_shape=jax.ShapeDtypeStruct((NP, OS), jnp.float32),
    )(agg, xp, dis, Wz, bz, LzT, Lz_b, Wh, bh, LhT, Lh_b, linT, lin_b)


def kernel(x, edge_index, edge_weight, Wz, bz, Lz_w, Lz_b, Wr, br, Lr_w, Lr_b,
           Wh, bh, Lh_w, Lh_b, att, lin_w, lin_b):
    src = edge_index[0]
    dst = edge_index[1]
    agg, dis = _sc_aggregate(x, src, dst, edge_weight)
    xp = jnp.pad(x, ((0, NP - N), (0, 0)))
    out = _tc_dense(agg, xp, dis,
                    Wz, bz, Lz_w[:, :OC].T, Lz_b,
                    Wh, bh, Lh_w[:, :OC].T, Lh_b,
                    lin_w.T, lin_b)
    return out[:N]


# speedup vs baseline: 29.6444x; 1.8818x over previous
import functools

import jax
import jax.numpy as jnp
from jax import lax
from jax.experimental import pallas as pl
from jax.experimental.pallas import tpu as pltpu
from jax.experimental.pallas import tpu_sc as plsc

N = 10000
E = 320000
F = 128
OC = 128
OS = 64

NC = 2
NS = 16
NW = NC * NS
NP = 10240
SL = NP // NS
EDEG = E // NS
EW_ = E // NW
C = 80
NCH = EW_ // C
L = 16

_MAGIC = 0x5F3759DF


def _rsqrt16(v):
    i = plsc.bitcast(v, jnp.int32)
    y = plsc.bitcast(jnp.int32(_MAGIC) - (i >> 1), jnp.float32)
    for _ in range(3):
        y = y * (1.5 - 0.5 * v * y * y)
    return y


PKW = 3 * C
NCHA = EDEG // C


def _sc_body(x_hbm, pk_hbm, agg_hbm, dis_hbm,
             rows0, rows1, pkd0, pkd1, dstc0, dstc1, ewc, normb, acc, disloc,
             degsh, dissh, aggsh,
             sempk0, sempk1, semg0, semg1, semsc0, semsc1):
    cid = lax.axis_index("c")
    sid = lax.axis_index("s")
    wid = sid * NC + cid
    zero16 = jnp.zeros((L,), jnp.float32)

    pkslot = (pkd0, pkd1)
    rowslot = (rows0, rows1)
    dstslot = (dstc0, dstc1)
    sempk = (sempk0, sempk1)
    semg = (semg0, semg1)
    semsc = (semsc0, semsc1)

    def pk_at(gc):
        return pk_hbm.at[pl.ds(gc * PKW, PKW)]

    def zrow(i, _):
        for k in range(F // L):
            rows0[i, pl.ds(k * L, L)] = zero16
        return 0
    lax.fori_loop(0, C, zrow, 0)

    def zacc(i, _):
        acc[pl.ds(i * L, L)] = zero16
        return 0
    lax.fori_loop(0, SL // L, zacc, 0)

    pltpu.sync_copy(acc, degsh.at[pl.ds(sid * SL, SL)])
    for j in range(SL // C):
        pltpu.sync_copy(rows0, aggsh.at[pl.ds(sid * SL + j * C, C), :])
    plsc.subcore_barrier()

    gc0 = sid * NCHA
    pltpu.async_copy(pk_at(gc0), pkd0, sempk0)
    pltpu.async_copy(pk_at(gc0 + 1), pkd1, sempk1)

    def dbody(i, _):
        for b in range(2):
            g = i * 2 + b
            pltpu.make_async_copy(pk_at(gc0 + g), pkslot[b], sempk[b]).wait()
            for k in range(C // L):
                dstc0[pl.ds(k * L, L)] = pkslot[b][pl.ds(C + k * L, L)]
                ewc[pl.ds(k * L, L)] = plsc.bitcast(
                    pkslot[b][pl.ds(2 * C + k * L, L)], jnp.float32)

            @pl.when(g + 2 < NCHA)
            def _():
                pltpu.async_copy(pk_at(gc0 + g + 2), pkslot[b], sempk[b])

            pltpu.sync_copy(ewc, degsh.at[dstc0], add=True)
        return 0
    lax.fori_loop(0, NCHA // 2, dbody, 0)
    plsc.subcore_barrier()

    pltpu.sync_copy(degsh.at[pl.ds(sid * SL, SL)], acc)

    def dstep(j, _):
        v = acc[pl.ds(j * L, L)] + 1.0
        acc[pl.ds(j * L, L)] = _rsqrt16(v)
        return 0
    lax.fori_loop(0, SL // L, dstep, 0)

    pltpu.sync_copy(acc, dissh.at[pl.ds(sid * SL, SL)])

    @pl.when(cid == 0)
    def _():
        pltpu.sync_copy(acc, dis_hbm.at[pl.ds(sid * SL, SL)])

    plsc.subcore_barrier()
    pltpu.sync_copy(dissh, disloc)

    cb0 = wid * NCH

    def norm_scale(b):
        pkd_b, rows_b, dstc_b = pkslot[b], rowslot[b], dstslot[b]
        for q in range(C // L):
            s16 = pkd_b[pl.ds(q * L, L)]
            d16 = pkd_b[pl.ds(C + q * L, L)]
            w16 = plsc.bitcast(pkd_b[pl.ds(2 * C + q * L, L)], jnp.float32)
            dis_s = plsc.load_gather(disloc, [s16])
            dis_d = plsc.load_gather(disloc, [d16])
            normb[pl.ds(q * L, L)] = dis_s * w16 * dis_d
            dstc_b[pl.ds(q * L, L)] = d16

        def scale(e, _):
            splat = plsc.load_gather(normb, [jnp.full((L,), e, jnp.int32)])
            for k in range(F // L):
                rows_b[e, pl.ds(k * L, L)] = rows_b[e, pl.ds(k * L, L)] * splat
            return 0
        lax.fori_loop(0, C, scale, 0)

    pltpu.async_copy(pk_at(cb0), pkd0, sempk0)
    pltpu.async_copy(pk_at(cb0 + 1), pkd1, sempk1)
    pltpu.make_async_copy(pk_at(cb0), pkd0, sempk0).wait()
    pltpu.async_copy(x_hbm.at[pkd0.at[pl.ds(0, C)]], rows0, semg0)

    def cbody(i, _):
        for b in range(2):
            o = 1 - b
            g = i * 2 + b
            pltpu.make_async_copy(
                x_hbm.at[pkslot[b].at[pl.ds(0, C)]], rowslot[b], semg[b]).wait()
            pltpu.make_async_copy(pk_at(cb0 + g + 1), pkslot[o], sempk[o]).wait()
            pltpu.async_copy(x_hbm.at[pkslot[o].at[pl.ds(0, C)]],
                             rowslot[o], semg[o])
            norm_scale(b)
            pltpu.sync_copy(rowslot[b], aggsh.at[dstslot[b]], add=True)
            if b == 0:
                pltpu.async_copy(pk_at(cb0 + g + 2), pkslot[b], sempk[b])
            else:
                @pl.when(g + 2 < NCH)
                def _():
                    pltpu.async_copy(pk_at(cb0 + g + 2), pkslot[b], sempk[b])
        return 0
    lax.fori_loop(0, NCH // 2, cbody, 0)

    pltpu.make_async_copy(
        x_hbm.at[pkd0.at[pl.ds(0, C)]], rows0, semg0).wait()
    norm_scale(0)
    pltpu.sync_copy(rows0, aggsh.at[dstc0], add=True)

    plsc.subcore_barrier()
    pltpu.sync_copy(aggsh.at[pl.ds(sid * SL, SL), :],
                    agg_hbm.at[cid, pl.ds(sid * SL, SL), :])


@jax.jit
def _sc_aggregate(x, pk):
    return pl.kernel(
        _sc_body,
        out_type=[
            jax.ShapeDtypeStruct((NC, NP, F), jnp.float32),
            jax.ShapeDtypeStruct((NP,), jnp.float32),
        ],
        mesh=plsc.VectorSubcoreMesh(core_axis_name="c", subcore_axis_name="s"),
        compiler_params=pltpu.CompilerParams(needs_layout_passes=False),
        scratch_types=[
            pltpu.VMEM((C, F), jnp.float32),
            pltpu.VMEM((C, F), jnp.float32),
            pltpu.VMEM((PKW,), jnp.int32),
            pltpu.VMEM((PKW,), jnp.int32),
            pltpu.VMEM((C,), jnp.int32),
            pltpu.VMEM((C,), jnp.int32),
            pltpu.VMEM((C,), jnp.float32),
            pltpu.VMEM((C,), jnp.float32),
            pltpu.VMEM((SL,), jnp.float32),
            pltpu.VMEM((NP,), jnp.float32),
            pltpu.VMEM_SHARED((NP,), jnp.float32),
            pltpu.VMEM_SHARED((NP,), jnp.float32),
            pltpu.VMEM_SHARED((NP, F), jnp.float32),
            pltpu.SemaphoreType.DMA,
            pltpu.SemaphoreType.DMA,
            pltpu.SemaphoreType.DMA,
            pltpu.SemaphoreType.DMA,
            pltpu.SemaphoreType.DMA,
            pltpu.SemaphoreType.DMA,
        ],
    )(x, pk)


BR = 256


def _tc_body(agg_ref, x_ref, dis_ref, wz_ref, bz_ref, lzt_ref, lzb_ref,
             wh_ref, bh_ref, lht_ref, lhb_ref, lint_ref, linb_ref, out_ref):
    dis = dis_ref[:]
    d2 = (dis * dis).reshape(BR, 1)
    a = agg_ref[0] + agg_ref[1] + x_ref[:] * d2
    tz = jnp.dot(a, wz_ref[:], preferred_element_type=jnp.float32) + bz_ref[:]
    z = jax.nn.sigmoid(
        jnp.dot(tz, lzt_ref[:], preferred_element_type=jnp.float32) + lzb_ref[:])
    th = jnp.dot(a, wh_ref[:], preferred_element_type=jnp.float32) + bh_ref[:]
    ht = jnp.tanh(
        jnp.dot(th, lht_ref[:], preferred_element_type=jnp.float32) + lhb_ref[:])
    h = jax.nn.relu((1.0 - z) * ht)
    out_ref[:, :] = (
        jnp.dot(h, lint_ref[:], preferred_element_type=jnp.float32) + linb_ref[:])


@jax.jit
def _tc_dense(agg, xp, dis, Wz, bz, LzT, Lz_b, Wh, bh, LhT, Lh_b, linT, lin_b):
    full = lambda shp: pl.BlockSpec(shp, lambda i: tuple(0 for _ in shp))
    return pl.pallas_call(
        _tc_body,
        grid=(NP // BR,),
        in_specs=[
            pl.BlockSpec((NC, BR, F), lambda i: (0, i, 0)),
            pl.BlockSpec((BR, F), lambda i: (i, 0)),
            pl.BlockSpec((BR,), lambda i: (i,)),
            full((F, OC)), full((OC,)), full((OC, OC)), full((OC,)),
            full((F, OC)), full((OC,)), full((OC, OC)), full((OC,)),
            full((OC, OS)), full((OS,)),
        ],
        out_specs=pl.BlockSpec((BR, OS), lambda i: (i, 0)),
        out_shape=jax.ShapeDtypeStruct((NP, OS), jnp.float32),
    )(agg, xp, dis, Wz, bz, LzT, Lz_b, Wh, bh, LhT, Lh_b, linT, lin_b)


def kernel(x, edge_index, edge_weight, Wz, bz, Lz_w, Lz_b, Wr, br, Lr_w, Lr_b,
           Wh, bh, Lh_w, Lh_b, att, lin_w, lin_b):
    src = edge_index[0]
    dst = edge_index[1]
    ew32 = jax.lax.bitcast_convert_type(edge_weight, jnp.int32)
    pk = jnp.stack([src.reshape(E // C, C), dst.reshape(E // C, C),
                    ew32.reshape(E // C, C)], axis=1).reshape(-1)
    agg, dis = _sc_aggregate(x, pk)
    xp = jnp.pad(x, ((0, NP - N), (0, 0)))
    out = _tc_dense(agg, xp, dis,
                    Wz, bz, Lz_w[:, :OC].T, Lz_b,
                    Wh, bh, Lh_w[:, :OC].T, Lh_b,
                    lin_w.T, lin_b)
    return out[:N]


# speedup vs baseline: 32.9790x; 1.1125x over previous
import functools

import jax
import jax.numpy as jnp
from jax import lax
from jax.experimental import pallas as pl
from jax.experimental.pallas import tpu as pltpu
from jax.experimental.pallas import tpu_sc as plsc

N = 10000
E = 320000
F = 128
OC = 128
OS = 64

NC = 2
NS = 16
NW = NC * NS
NP = 10240
SL = NP // NS
EDEG = E // NS
EW_ = E // NW
C = 80
NCH = EW_ // C
L = 16

_MAGIC = 0x5F3759DF


def _rsqrt16(v):
    i = plsc.bitcast(v, jnp.int32)
    y = plsc.bitcast(jnp.int32(_MAGIC) - (i >> 1), jnp.float32)
    for _ in range(3):
        y = y * (1.5 - 0.5 * v * y * y)
    return y


PKW = 3 * C
NCHA = EDEG // C
AB = 5


def _sc_body(x_hbm, pk_hbm, agg_hbm, dis_hbm,
             rows0, rows1, pkd0, pkd1, pka0, pka1, dstc0, dstc1,
             ewc, normb, acc, disloc,
             degsh, dissh, aggsh,
             sempk0, sempk1, sempka0, sempka1, semg0, semg1):
    cid = lax.axis_index("c")
    sid = lax.axis_index("s")
    wid = sid * NC + cid
    zero16 = jnp.zeros((L,), jnp.float32)

    pkslot = (pkd0, pkd1)
    pkaslot = (pka0, pka1)
    rowslot = (rows0, rows1)
    dstslot = (dstc0, dstc1)
    sempk = (sempk0, sempk1)
    sempka = (sempka0, sempka1)
    semg = (semg0, semg1)

    def pk_at(gc, nch=1):
        return pk_hbm.at[pl.ds(gc * PKW, nch * PKW)]

    def zrow(i, _):
        for k in range(F // L):
            rows0[i, pl.ds(k * L, L)] = zero16
        return 0
    lax.fori_loop(0, C, zrow, 0)

    def zacc(i, _):
        acc[pl.ds(i * L, L)] = zero16
        return 0
    lax.fori_loop(0, SL // L, zacc, 0)

    pltpu.sync_copy(acc, degsh.at[pl.ds(sid * SL, SL)])
    for j in range(SL // C):
        pltpu.sync_copy(rows0, aggsh.at[pl.ds(sid * SL + j * C, C), :])
    plsc.subcore_barrier()

    gc0 = sid * NCHA
    pltpu.async_copy(pk_at(gc0, AB), pka0, sempka0)
    pltpu.async_copy(pk_at(gc0 + AB, AB), pka1, sempka1)

    def dbody(i, _):
        for b in range(2):
            t = i * 2 + b
            pltpu.make_async_copy(
                pk_at(gc0 + t * AB, AB), pkaslot[b], sempka[b]).wait()
            for sub in range(AB):
                for k in range(C // L):
                    dstc0[pl.ds(k * L, L)] = (
                        pkaslot[b][pl.ds(sub * PKW + C + k * L, L)])
                    ewc[pl.ds(k * L, L)] = plsc.bitcast(
                        pkaslot[b][pl.ds(sub * PKW + 2 * C + k * L, L)],
                        jnp.float32)
                pltpu.sync_copy(ewc, degsh.at[dstc0], add=True)

            @pl.when(t + 2 < NCHA // AB)
            def _():
                pltpu.async_copy(
                    pk_at(gc0 + (t + 2) * AB, AB), pkaslot[b], sempka[b])
        return 0
    lax.fori_loop(0, NCHA // AB // 2, dbody, 0)
    plsc.subcore_barrier()

    pltpu.sync_copy(degsh.at[pl.ds(sid * SL, SL)], acc)

    def dstep(j, _):
        v = acc[pl.ds(j * L, L)] + 1.0
        acc[pl.ds(j * L, L)] = _rsqrt16(v)
        return 0
    lax.fori_loop(0, SL // L, dstep, 0)

    pltpu.sync_copy(acc, dissh.at[pl.ds(sid * SL, SL)])

    @pl.when(cid == 0)
    def _():
        pltpu.sync_copy(acc, dis_hbm.at[pl.ds(sid * SL, SL)])

    plsc.subcore_barrier()
    pltpu.sync_copy(dissh, disloc)

    cb0 = wid * NCH

    def norm_scale(b):
        pkd_b, rows_b, dstc_b = pkslot[b], rowslot[b], dstslot[b]
        for q in range(C // L):
            s16 = pkd_b[pl.ds(q * L, L)]
            d16 = pkd_b[pl.ds(C + q * L, L)]
            w16 = plsc.bitcast(pkd_b[pl.ds(2 * C + q * L, L)], jnp.float32)
            dis_s = plsc.load_gather(disloc, [s16])
            dis_d = plsc.load_gather(disloc, [d16])
            normb[pl.ds(q * L, L)] = dis_s * w16 * dis_d
            dstc_b[pl.ds(q * L, L)] = d16

        def scale(e, _):
            splat = plsc.load_gather(normb, [jnp.full((L,), e, jnp.int32)])
            for k in range(F // L):
                rows_b[e, pl.ds(k * L, L)] = rows_b[e, pl.ds(k * L, L)] * splat
            return 0
        lax.fori_loop(0, C, scale, 0)

    pltpu.async_copy(pk_at(cb0), pkd0, sempk0)
    pltpu.async_copy(pk_at(cb0 + 1), pkd1, sempk1)
    pltpu.make_async_copy(pk_at(cb0), pkd0, sempk0).wait()
    pltpu.async_copy(x_hbm.at[pkd0.at[pl.ds(0, C)]], rows0, semg0)

    def cbody(i, _):
        for b in range(2):
            o = 1 - b
            g = i * 2 + b
            pltpu.make_async_copy(
                x_hbm.at[pkslot[b].at[pl.ds(0, C)]], rowslot[b], semg[b]).wait()
            pltpu.make_async_copy(pk_at(cb0 + g + 1), pkslot[o], sempk[o]).wait()
            pltpu.async_copy(x_hbm.at[pkslot[o].at[pl.ds(0, C)]],
                             rowslot[o], semg[o])
            norm_scale(b)
            pltpu.sync_copy(rowslot[b], aggsh.at[dstslot[b]], add=True)
            if b == 0:
                pltpu.async_copy(pk_at(cb0 + g + 2), pkslot[b], sempk[b])
            else:
                @pl.when(g + 2 < NCH)
                def _():
                    pltpu.async_copy(pk_at(cb0 + g + 2), pkslot[b], sempk[b])
        return 0
    lax.fori_loop(0, NCH // 2, cbody, 0)

    pltpu.make_async_copy(
        x_hbm.at[pkd0.at[pl.ds(0, C)]], rows0, semg0).wait()
    norm_scale(0)
    pltpu.sync_copy(rows0, aggsh.at[dstc0], add=True)

    plsc.subcore_barrier()
    pltpu.sync_copy(aggsh.at[pl.ds(sid * SL, SL), :],
                    agg_hbm.at[cid, pl.ds(sid * SL, SL), :])


@jax.jit
def _sc_aggregate(x, pk):
    return pl.kernel(
        _sc_body,
        out_type=[
            jax.ShapeDtypeStruct((NC, NP, F), jnp.float32),
            jax.ShapeDtypeStruct((NP,), jnp.float32),
        ],
        mesh=plsc.VectorSubcoreMesh(core_axis_name="c", subcore_axis_name="s"),
        compiler_params=pltpu.CompilerParams(needs_layout_passes=False),
        scratch_types=(
            [pltpu.VMEM((C, F), jnp.float32)] * 2
            + [pltpu.VMEM((PKW,), jnp.int32)] * 2
            + [pltpu.VMEM((AB * PKW,), jnp.int32)] * 2
            + [pltpu.VMEM((C,), jnp.int32)] * 2
            + [
                pltpu.VMEM((C,), jnp.float32),
                pltpu.VMEM((C,), jnp.float32),
                pltpu.VMEM((SL,), jnp.float32),
                pltpu.VMEM((NP,), jnp.float32),
                pltpu.VMEM_SHARED((NP,), jnp.float32),
                pltpu.VMEM_SHARED((NP,), jnp.float32),
                pltpu.VMEM_SHARED((NP, F), jnp.float32),
            ]
            + [pltpu.SemaphoreType.DMA] * 6
        ),
    )(x, pk)


BR = 400


def _tc_body(agg_ref, x_ref, dis_ref, wz_ref, bz_ref, lzt_ref, lzb_ref,
             wh_ref, bh_ref, lht_ref, lhb_ref, lint_ref, linb_ref, out_ref):
    dis = dis_ref[:]
    d2 = dis * dis
    a = agg_ref[0] + agg_ref[1] + x_ref[:] * d2
    tz = jnp.dot(a, wz_ref[:], preferred_element_type=jnp.float32) + bz_ref[:]
    z = jax.nn.sigmoid(
        jnp.dot(tz, lzt_ref[:], preferred_element_type=jnp.float32) + lzb_ref[:])
    th = jnp.dot(a, wh_ref[:], preferred_element_type=jnp.float32) + bh_ref[:]
    ht = jnp.tanh(
        jnp.dot(th, lht_ref[:], preferred_element_type=jnp.float32) + lhb_ref[:])
    h = jax.nn.relu((1.0 - z) * ht)
    out_ref[:, :] = (
        jnp.dot(h, lint_ref[:], preferred_element_type=jnp.float32) + linb_ref[:])


@jax.jit
def _tc_dense(agg, xp, dis, Wz, bz, LzT, Lz_b, Wh, bh, LhT, Lh_b, linT, lin_b):
    full = lambda shp: pl.BlockSpec(shp, lambda i: tuple(0 for _ in shp))
    return pl.pallas_call(
        _tc_body,
        grid=(N // BR,),
        in_specs=[
            pl.BlockSpec((NC, BR, F), lambda i: (0, i, 0)),
            pl.BlockSpec((BR, F), lambda i: (i, 0)),
            pl.BlockSpec((BR, 1), lambda i: (i, 0)),
            full((F, OC)), full((OC,)), full((OC, OC)), full((OC,)),
            full((F, OC)), full((OC,)), full((OC, OC)), full((OC,)),
            full((OC, OS)), full((OS,)),
        ],
        out_specs=pl.BlockSpec((BR, OS), lambda i: (i, 0)),
        out_shape=jax.ShapeDtypeStruct((N, OS), jnp.float32),
    )(agg, xp, dis, Wz, bz, LzT, Lz_b, Wh, bh, LhT, Lh_b, linT, lin_b)


def kernel(x, edge_index, edge_weight, Wz, bz, Lz_w, Lz_b, Wr, br, Lr_w, Lr_b,
           Wh, bh, Lh_w, Lh_b, att, lin_w, lin_b):
    src = edge_index[0]
    dst = edge_index[1]
    ew32 = jax.lax.bitcast_convert_type(edge_weight, jnp.int32)
    pk = jnp.stack([src.reshape(E // C, C), dst.reshape(E // C, C),
                    ew32.reshape(E // C, C)], axis=1).reshape(-1)
    agg, dis = _sc_aggregate(x, pk)
    return _tc_dense(agg, x, dis.reshape(NP, 1),
                     Wz, bz, Lz_w[:, :OC].T, Lz_b,
                     Wh, bh, Lh_w[:, :OC].T, Lh_b,
                     lin_w.T, lin_b)
